# baseline (speedup 1.0000x reference)
import functools

import jax
import jax.numpy as jnp
from jax import lax
from jax.experimental import pallas as pl
from jax.experimental.pallas import tpu as pltpu
from jax.experimental.pallas import tpu_sc as plsc

NC, NS = 2, 16
NW = NC * NS
NPAD = 10240
ZR = NPAD // NS
HW = 16
F = 32
K = 500


def _sc_mesh():
    return plsc.VectorSubcoreMesh(
        core_axis_name="c", subcore_axis_name="s", num_cores=NC, num_subcores=NS
    )


_SC_PARAMS = pltpu.CompilerParams(use_tc_tiling_on_sc=False)


def _hist(ei3):
    E = ei3.shape[1] * ei3.shape[2]
    EPW = E // NW
    CH = EPW // K

    @functools.partial(
        pl.kernel,
        out_type=[
            jax.ShapeDtypeStruct((NC, NPAD, F), jnp.float32),
            jax.ShapeDtypeStruct((NC, NPAD, F), jnp.float32),
        ],
        mesh=_sc_mesh(),
        compiler_params=_SC_PARAMS,
        scratch_types=[
            pltpu.VMEM((CH, K), jnp.int32),
            pltpu.VMEM((CH, K), jnp.int32),
            pltpu.VMEM((K, HW), jnp.float32),
            pltpu.VMEM((ZR, HW), jnp.float32),
            pltpu.VMEM((ZR, F), jnp.float32),
            pltpu.VMEM_SHARED((NPAD, HW), jnp.float32),
            pltpu.VMEM_SHARED((NPAD, HW), jnp.float32),
            pltpu.SemaphoreType.DMA,
            pltpu.SemaphoreType.DMA,
        ],
    )
    def k(ei_h, out_o, out_i, idx_a, idx_b, ones, zbuf, wide,
          acc_o, acc_i, sem_o, sem_i):
        cid = lax.axis_index("c")
        sid = lax.axis_index("s")
        wid = sid * NC + cid

        def fill(i, _):
            ones[i, :] = jnp.ones((HW,), jnp.float32)
            zbuf[i % ZR, :] = jnp.zeros((HW,), jnp.float32)
            return 0

        lax.fori_loop(0, K, fill, 0)
        pltpu.sync_copy(zbuf, acc_o.at[pl.ds(sid * ZR, ZR)])
        pltpu.sync_copy(zbuf, acc_i.at[pl.ds(sid * ZR, ZR)])
        rb = wid * CH
        pltpu.sync_copy(ei_h.at[0, pl.ds(rb, CH)], idx_a)
        pltpu.sync_copy(ei_h.at[1, pl.ds(rb, CH)], idx_b)
        plsc.subcore_barrier()

        so = [None] * CH
        si = [None] * CH
        for i in range(CH):
            so[i] = pltpu.async_copy(ones, acc_o.at[idx_a.at[i]], sem_o,
                                     add=True)
            si[i] = pltpu.async_copy(ones, acc_i.at[idx_b.at[i]], sem_i,
                                     add=True)
        for i in range(CH):
            so[i].wait()
            si[i].wait()
        plsc.subcore_barrier()

        for acc, out in ((acc_o, out_o), (acc_i, out_i)):
            pltpu.sync_copy(acc.at[pl.ds(sid * ZR, ZR)], zbuf)

            def dup(r, _):
                v = zbuf[r, :]
                wide[r, pl.ds(0, HW)] = v
                wide[r, pl.ds(HW, HW)] = v
                return 0

            lax.fori_loop(0, ZR, dup, 0)
            pltpu.sync_copy(wide, out.at[cid, pl.ds(sid * ZR, ZR)])

    return k(ei3)


def _edge_agg(table, ei3):
    E = ei3.shape[1] * ei3.shape[2]
    EPW = E // NW
    CH = EPW // K

    @functools.partial(
        pl.kernel,
        out_type=jax.ShapeDtypeStruct((NC, NPAD, F), jnp.float32),
        mesh=_sc_mesh(),
        compiler_params=_SC_PARAMS,
        scratch_types=[
            pltpu.VMEM((CH, K), jnp.int32),
            pltpu.VMEM((CH, K), jnp.int32),
            pltpu.VMEM((K, F), jnp.float32),
            pltpu.VMEM((K, F), jnp.float32),
            pltpu.VMEM((K, F), jnp.float32),
            pltpu.VMEM_SHARED((NPAD, F), jnp.float32),
            pltpu.SemaphoreType.DMA,
            pltpu.SemaphoreType.DMA,
        ],
    )
    def k(table_h, ei_h, out_h, idx_s, idx_d, rows0, rows1, rows2,
          acc, gsem, ssem):
        cid = lax.axis_index("c")
        sid = lax.axis_index("s")
        wid = sid * NC + cid
        rows = [rows0, rows1, rows2]

        def zfill(i, _):
            rows0[i, pl.ds(0, 16)] = jnp.zeros((16,), jnp.float32)
            rows0[i, pl.ds(16, 16)] = jnp.zeros((16,), jnp.float32)
            return 0

        lax.fori_loop(0, ZR, zfill, 0)
        pltpu.sync_copy(rows0.at[pl.ds(0, ZR)], acc.at[pl.ds(sid * ZR, ZR)])
        rb = wid * CH
        pltpu.sync_copy(ei_h.at[0, pl.ds(rb, CH)], idx_s)
        pltpu.sync_copy(ei_h.at[1, pl.ds(rb, CH)], idx_d)
        plsc.subcore_barrier()

        g = [None] * CH
        s = [None] * CH
        g[0] = pltpu.async_copy(table_h.at[idx_s.at[0]], rows[0], gsem)
        if CH >= 2:
            g[1] = pltpu.async_copy(table_h.at[idx_s.at[1]], rows[1], gsem)
        for i in range(CH):
            g[i].wait()
            s[i] = pltpu.async_copy(rows[i % 3], acc.at[idx_d.at[i]], ssem,
                                    add=True)
            if i + 2 < CH:
                if i >= 1:
                    s[i - 1].wait()
                g[i + 2] = pltpu.async_copy(table_h.at[idx_s.at[i + 2]],
                                            rows[(i + 2) % 3], gsem)
        if CH >= 3:
            s[CH - 3].wait()
        if CH >= 2:
            s[CH - 2].wait()
        s[CH - 1].wait()
        plsc.subcore_barrier()
        pltpu.sync_copy(acc.at[pl.ds(sid * ZR, ZR)],
                        out_h.at[cid, pl.ds(sid * ZR, ZR)])

    return k(table, ei3)


def _prep(xp, w1bd, dop, ddp):
    RP = xp.shape[0]
    RN = dop.shape[1]

    def body(x_ref, w_ref, do_ref, di_ref, t_ref, no_ref, ni_ref):
        do = do_ref[0] + do_ref[1]
        di = di_ref[0] + di_ref[1]
        no = jnp.where(do > 0, lax.rsqrt(jnp.maximum(do, 1e-12)), 0.0)
        ni = jnp.where(di > 0, lax.rsqrt(jnp.maximum(di, 1e-12)), 0.0)
        no_ref[...] = no
        ni_ref[...] = ni
        y = jnp.dot(x_ref[...], w_ref[...], preferred_element_type=jnp.float32)
        t_ref[...] = y * no[:RP]

    return pl.pallas_call(
        body,
        out_shape=[
            jax.ShapeDtypeStruct((RP, 128), jnp.float32),
            jax.ShapeDtypeStruct((RN, 128), jnp.float32),
            jax.ShapeDtypeStruct((RN, 128), jnp.float32),
        ],
    )(xp, w1bd, dop, ddp)


def _mid(parts, nip, nop, bp, wbd):
    RP = NPAD // 4 - (NPAD - 10000) // 4

    def body(p_ref, ni_ref, no_ref, b_ref, w_ref, t_ref):
        agg = p_ref[0, :RP] + p_ref[1, :RP]
        b = b_ref[...]
        bp = jnp.concatenate([b, b, b, b])
        h = jnp.maximum(agg * ni_ref[:RP] + bp, 0.0)
        y = jnp.dot(h, w_ref[...], preferred_element_type=jnp.float32)
        t_ref[...] = y * no_ref[:RP]

    return pl.pallas_call(
        body,
        out_shape=jax.ShapeDtypeStruct((RP, 128), jnp.float32),
    )(parts, nip, nop, bp, wbd)


def _final(parts, nip, bp, Wfc, bfc):
    RP = 2500

    def body(p_ref, ni_ref, b_ref, w_ref, bf_ref, o_ref):
        agg = p_ref[0, :RP] + p_ref[1, :RP]
        b = b_ref[...]
        bp = jnp.concatenate([b, b, b, b])
        h = jnp.maximum(agg * ni_ref[:RP] + bp, 0.0)
        m = jnp.max(h, axis=0, keepdims=True)
        m32 = jnp.maximum(jnp.maximum(m[:, 0:32], m[:, 32:64]),
                          jnp.maximum(m[:, 64:96], m[:, 96:128]))
        logits = jnp.dot(m32, w_ref[...],
                         preferred_element_type=jnp.float32) + bf_ref[...]
        mx = jnp.max(logits, axis=-1, keepdims=True)
        e = jnp.exp(logits - mx)
        o_ref[...] = e / jnp.sum(e, axis=-1, keepdims=True)

    return pl.pallas_call(
        body,
        out_shape=jax.ShapeDtypeStruct((1, bfc.shape[-1]), jnp.float32),
    )(parts, nip, bp, Wfc, bfc)


def kernel(inputs, edge_index, W1, b1, W2, b2, W3, b3, Wfc, bfc):
    N = inputs.shape[0]
    E = edge_index.shape[1]
    ei3 = edge_index.reshape(2, E // K, K)
    xp = inputs.reshape(N // 4, 512)
    eye4 = jnp.eye(4, dtype=jnp.float32)
    w1bd = jnp.kron(eye4, W1)
    w2bd = jnp.kron(eye4, W2)
    w3bd = jnp.kron(eye4, W3)
    bfc2 = bfc.reshape(1, -1)

    dop, ddp = _hist(ei3)
    dopP = dop.reshape(NC, NPAD // 4, 128)
    ddpP = ddp.reshape(NC, NPAD // 4, 128)
    t1p, nop, nip = _prep(xp, w1bd, dopP, ddpP)

    p1 = _edge_agg(t1p.reshape(N, F), ei3).reshape(NC, NPAD // 4, 128)
    t2p = _mid(p1, nip, nop, b1, w2bd)
    p2 = _edge_agg(t2p.reshape(N, F), ei3).reshape(NC, NPAD // 4, 128)
    t3p = _mid(p2, nip, nop, b2, w3bd)
    p3 = _edge_agg(t3p.reshape(N, F), ei3).reshape(NC, NPAD // 4, 128)
    return _final(p3, nip, b3, Wfc, bfc2)
